# Initial kernel scaffold; baseline (speedup 1.0000x reference)
import jax
import jax.numpy as jnp
from jax.experimental import pallas as pl


def kernel(hidden_preactivation_BH):
    raise NotImplementedError("write your pallas kernel here")


# speedup vs baseline: 4.0159x; 4.0159x over previous
import jax
import jax.numpy as jnp
from jax.experimental import pallas as pl
from jax.experimental.pallas import tpu as pltpu

K = 64
ROWS_PER_BLOCK = 8


def _body(x_ref, o_ref):
    x = x_ref[...]
    u = jax.lax.bitcast_convert_type(x, jnp.int32)
    key = jnp.where(u >= 0, u, u ^ jnp.int32(0x7FFFFFFF))

    r = x.shape[0]
    lo = jnp.full((r, 1), -2147483648, dtype=jnp.int32)
    hi = jnp.full((r, 1), 2147483647, dtype=jnp.int32)

    def it(_, carry):
        lo, hi = carry
        mid = (lo >> 1) + (hi >> 1) + ((lo & 1) | (hi & 1))
        cnt = jnp.sum((key >= mid).astype(jnp.int32), axis=1, keepdims=True)
        pred = cnt >= K
        lo = jnp.where(pred, mid, lo)
        hi = jnp.where(pred, hi, mid - jnp.int32(1))
        return lo, hi

    lo, hi = jax.lax.fori_loop(0, 32, it, (lo, hi))
    v = lo

    gt = key > v
    cnt_gt = jnp.sum(gt.astype(jnp.int32), axis=1, keepdims=True)
    ties_to_keep = K - cnt_gt
    tie = key == v
    tie_rank = tie.astype(jnp.int32)
    h = x.shape[1]
    s = 1
    while s < h:
        shifted = jnp.concatenate(
            [jnp.zeros((r, s), jnp.int32), tie_rank[:, : h - s]], axis=1
        )
        tie_rank = tie_rank + shifted
        s *= 2
    keep = gt | (tie & (tie_rank <= ties_to_keep))
    o_ref[...] = jnp.where(keep, x, jnp.float32(0))


def kernel(hidden_preactivation_BH):
    b, h = hidden_preactivation_BH.shape
    grid = b // ROWS_PER_BLOCK
    return pl.pallas_call(
        _body,
        grid=(grid,),
        in_specs=[pl.BlockSpec((ROWS_PER_BLOCK, h), lambda i: (i, 0))],
        out_specs=pl.BlockSpec((ROWS_PER_BLOCK, h), lambda i: (i, 0)),
        out_shape=jax.ShapeDtypeStruct((b, h), jnp.float32),
        compiler_params=pltpu.CompilerParams(
            dimension_semantics=("parallel",),
        ),
    )(hidden_preactivation_BH)


# speedup vs baseline: 4.6925x; 1.1685x over previous
import jax
import jax.numpy as jnp
import numpy as np
from jax import lax
from jax.experimental import pallas as pl
from jax.experimental.pallas import tpu as pltpu
from jax.experimental.pallas import tpu_sc as plsc

K = 64
B = 128
H = 32768
NC, NS, L = 2, 16, 16
NW = NC * NS
ROWS_PER_W = B // NW
NVREG = H // L
HEAD_VREGS = 16
CHUNK = 127
NCHUNK = (NVREG - HEAD_VREGS) // CHUNK
REBUILD_AT = 448
CAP = 2560

INT_MIN = np.int32(-2147483648)
MANT = np.int32(0x7FFFFFFF)


def _iota():
    return lax.iota(jnp.int32, L)


def _pcount(mask):
    return jnp.sum(mask.astype(jnp.int32))


def _keys_of(v):
    u = plsc.bitcast(v, jnp.int32)
    return u ^ ((u >> 31) & MANT)


def _axidx(name):
    return lax.axis_index(name)


def _body(x_hbm, o_hbm, buf, zbuf, cand_i, cand_k):
    wid = _axidx("c") * NS + _axidx("s")
    zeros = jnp.zeros((L,), jnp.float32)

    def _z(j, _):
        zbuf[pl.ds(j * L, L)] = zeros
        return 0
    lax.fori_loop(0, NVREG, _z, 0)

    def rebuild(cnt):
        mv = (cnt + L - 1) // L

        def keys_j(j, _):
            idx = cand_i[pl.ds(j * L, L)]
            valid = _iota() < (cnt - j * L)
            idxs = jnp.where(valid, idx, 0)
            v = plsc.load_gather(buf, [idxs])
            k = jnp.where(valid, _keys_of(v), INT_MIN)
            cand_k[pl.ds(j * L, L)] = k
            return 0
        lax.fori_loop(0, mv, keys_j, 0)

        def search_it(_, lohi):
            lo, hi = lohi
            mid = (lo >> 1) + (hi >> 1) + ((lo & 1) | (hi & 1))

            def cj(j, acc):
                k = cand_k[pl.ds(j * L, L)]
                return acc + (k >= mid).astype(jnp.int32)
            acc = lax.fori_loop(0, mv, cj, jnp.zeros((L,), jnp.int32))
            pred = jnp.sum(acc) >= K
            lo = jnp.where(pred, mid, lo)
            hi = jnp.where(pred, hi, mid - jnp.int32(1))
            return lo, hi
        v_key, _ = lax.fori_loop(
            0, 32, search_it, (INT_MIN, jnp.int32(2147483647)))

        def gj(j, acc):
            k = cand_k[pl.ds(j * L, L)]
            return acc + (k > v_key).astype(jnp.int32)
        cnt_gt = jnp.sum(lax.fori_loop(0, mv, gj, jnp.zeros((L,), jnp.int32)))
        allow = jnp.int32(K) - cnt_gt

        def comp_j(j, carry):
            oc, tr = carry
            k = cand_k[pl.ds(j * L, L)]
            idx = cand_i[pl.ds(j * L, L)]
            gt = k > v_key
            tie = k == v_key
            tp = plsc.cumsum(tie.astype(jnp.int32)) + tr
            keep = gt | (tie & (tp <= allow))
            plsc.store_compressed(cand_i.at[pl.ds(oc, L)], idx, mask=keep)
            return oc + _pcount(keep), tr + _pcount(tie)
        lax.fori_loop(0, mv, comp_j, (jnp.int32(0), jnp.int32(0)))

        thr_bits = jnp.full((L,), v_key ^ ((v_key >> 31) & MANT), jnp.int32)
        return plsc.bitcast(thr_bits, jnp.float32)

    def process_row(t, _):
        row = wid * ROWS_PER_W + t
        pltpu.sync_copy(x_hbm.at[row], buf)

        def seed_j(j, _):
            cand_i[pl.ds(j * L, L)] = _iota() + j * L
            return 0
        lax.fori_loop(0, HEAD_VREGS, seed_j, 0)
        thr = rebuild(jnp.int32(HEAD_VREGS * L))
        cnt = jnp.int32(K)

        def chunk_c(c, carry):
            cnt, thr = carry
            base = HEAD_VREGS + c * CHUNK

            def fil_j(j, cnt):
                vr = base + j
                v = buf[pl.ds(vr * L, L)]
                m = v > thr
                plsc.store_compressed(
                    cand_i.at[pl.ds(cnt, L)], _iota() + vr * L, mask=m)
                return cnt + _pcount(m)
            cnt = lax.fori_loop(0, CHUNK, fil_j, cnt)

            def do_rb(cnt):
                return jnp.int32(K), rebuild(cnt)

            cnt, thr = lax.cond(
                cnt > REBUILD_AT, do_rb, lambda c: (c, thr), cnt)
            return cnt, thr
        cnt, thr = lax.fori_loop(0, NCHUNK, chunk_c, (cnt, thr))

        rebuild(cnt)

        for j in range(K // L):
            idx = cand_i[pl.ds(j * L, L)]
            vals = plsc.load_gather(buf, [idx])
            plsc.store_scatter(zbuf, [idx], vals)
        pltpu.sync_copy(zbuf, o_hbm.at[row])
        for j in range(K // L):
            idx = cand_i[pl.ds(j * L, L)]
            plsc.store_scatter(zbuf, [idx], zeros)
        return 0

    lax.fori_loop(0, ROWS_PER_W, process_row, 0)


@jax.jit
def kernel(hidden_preactivation_BH):
    mesh = plsc.VectorSubcoreMesh(
        core_axis_name="c", subcore_axis_name="s",
        num_cores=NC, num_subcores=NS)
    return pl.kernel(
        _body,
        out_type=jax.ShapeDtypeStruct((B, H), jnp.float32),
        mesh=mesh,
        scratch_types=[
            pltpu.VMEM((H,), jnp.float32),
            pltpu.VMEM((H,), jnp.float32),
            pltpu.VMEM((CAP,), jnp.int32),
            pltpu.VMEM((CAP,), jnp.int32),
        ],
        compiler_params=pltpu.CompilerParams(needs_layout_passes=False),
    )(hidden_preactivation_BH)


# speedup vs baseline: 10.4557x; 2.2281x over previous
import functools

import jax
import jax.numpy as jnp
import numpy as np
from jax import lax
from jax.experimental import pallas as pl
from jax.experimental.pallas import tpu as pltpu
from jax.experimental.pallas import tpu_sc as plsc

K = 64
B = 128
H = 32768
NC, NS, L = 2, 16, 16
NW = NC * NS
ROWS_PER_W = B // NW
NVREG = H // L
HEAD_VREGS = 16
CHUNK = 127
NCHUNK = (NVREG - HEAD_VREGS) // CHUNK
REBUILD_AT = 448
CAP = 2560

INT_MIN = np.int32(-2147483648)
MANT = np.int32(0x7FFFFFFF)

parallel_loop = plsc.parallel_loop


def _iota():
    return lax.iota(jnp.int32, L)


def _pcount(mask):
    return plsc.all_reduce_population_count(mask)[0]


def _keys_of(v):
    u = plsc.bitcast(v, jnp.int32)
    return u ^ ((u >> 31) & MANT)


def _axidx(name):
    return lax.axis_index(name)


def _body(x_hbm, o_hbm, buf, zbuf, cand_i, cand_k):
    wid = _axidx("c") * NS + _axidx("s")
    zeros = jnp.zeros((L,), jnp.float32)

    @parallel_loop(0, H, L, unroll=8, carry=jnp.int32(0))
    def _z(off, c):
        zbuf[pl.ds(off, L)] = zeros
        return c

    def rebuild(cnt):
        mv = (cnt + L - 1) // L

        @parallel_loop(0, mv * L, L, unroll=2, carry=jnp.int32(0))
        def _keys(off, c):
            idx = cand_i[pl.ds(off, L)]
            valid = (_iota() + off) < cnt
            idxs = jnp.where(valid, idx, 0)
            v = plsc.load_gather(buf, [idxs])
            k = jnp.where(valid, _keys_of(v), INT_MIN)
            cand_k[pl.ds(off, L)] = k
            return c

        def search_it(_, lohi):
            lo, hi = lohi
            mid = (lo >> 1) + (hi >> 1) + ((lo & 1) | (hi & 1))

            @parallel_loop(0, mv * L, L, unroll=4,
                           carry=jnp.zeros((L,), jnp.int32))
            def acc(off, a):
                k = cand_k[pl.ds(off, L)]
                return a + (k >= mid).astype(jnp.int32)
            pred = jnp.sum(acc) >= K
            lo = jnp.where(pred, mid, lo)
            hi = jnp.where(pred, hi, mid - jnp.int32(1))
            return lo, hi
        v_key, _ = lax.fori_loop(
            0, 32, search_it, (INT_MIN, jnp.int32(2147483647)))

        @parallel_loop(0, mv * L, L, unroll=4,
                       carry=jnp.zeros((L,), jnp.int32))
        def gacc(off, a):
            k = cand_k[pl.ds(off, L)]
            return a + (k > v_key).astype(jnp.int32)
        cnt_gt = jnp.sum(gacc)
        allow = jnp.int32(K) - cnt_gt

        def comp_j(j, carry):
            oc, tr = carry
            k = cand_k[pl.ds(j * L, L)]
            idx = cand_i[pl.ds(j * L, L)]
            gt = k > v_key
            tie = k == v_key
            tp = plsc.cumsum(tie.astype(jnp.int32)) + tr
            keep = gt | (tie & (tp <= allow))
            plsc.store_compressed(cand_i.at[pl.ds(oc, L)], idx, mask=keep)
            return oc + _pcount(keep), tr + _pcount(tie)
        lax.fori_loop(0, mv, comp_j, (jnp.int32(0), jnp.int32(0)))

        thr_bits = jnp.full((L,), v_key ^ ((v_key >> 31) & MANT), jnp.int32)
        return plsc.bitcast(thr_bits, jnp.float32)

    def process_row(t, _):
        row = wid * ROWS_PER_W + t
        pltpu.sync_copy(x_hbm.at[row], buf)

        @parallel_loop(0, HEAD_VREGS * L, L, unroll=4, carry=jnp.int32(0))
        def _seed(off, c):
            cand_i[pl.ds(off, L)] = _iota() + off
            return c
        thr = rebuild(jnp.int32(HEAD_VREGS * L))
        cnt = jnp.int32(K)

        def chunk_c(c, carry):
            cnt, thr = carry
            base = HEAD_VREGS + c * CHUNK

            @parallel_loop(base * L, (base + CHUNK) * L, L,
                           unroll=4, carry=cnt)
            def cnt(off, cnt):
                v = buf[pl.ds(off, L)]
                m = v > thr
                plsc.store_compressed(
                    cand_i.at[pl.ds(cnt, L)], _iota() + off, mask=m)
                return cnt + _pcount(m)

            def do_rb(cnt):
                return jnp.int32(K), rebuild(cnt)

            cnt, thr = lax.cond(
                cnt > REBUILD_AT, do_rb, lambda c: (c, thr), cnt)
            return cnt, thr
        cnt, thr = lax.fori_loop(0, NCHUNK, chunk_c, (cnt, thr))

        rebuild(cnt)

        for j in range(K // L):
            idx = cand_i[pl.ds(j * L, L)]
            vals = plsc.load_gather(buf, [idx])
            plsc.store_scatter(zbuf, [idx], vals)
        pltpu.sync_copy(zbuf, o_hbm.at[row])
        for j in range(K // L):
            idx = cand_i[pl.ds(j * L, L)]
            plsc.store_scatter(zbuf, [idx], zeros)
        return 0

    lax.fori_loop(0, ROWS_PER_W, process_row, 0)


@jax.jit
def kernel(hidden_preactivation_BH):
    mesh = plsc.VectorSubcoreMesh(
        core_axis_name="c", subcore_axis_name="s",
        num_cores=NC, num_subcores=NS)
    return pl.kernel(
        _body,
        out_type=jax.ShapeDtypeStruct((B, H), jnp.float32),
        mesh=mesh,
        scratch_types=[
            pltpu.VMEM((H,), jnp.float32),
            pltpu.VMEM((H,), jnp.float32),
            pltpu.VMEM((CAP,), jnp.int32),
            pltpu.VMEM((CAP,), jnp.int32),
        ],
        compiler_params=pltpu.CompilerParams(needs_layout_passes=False),
    )(hidden_preactivation_BH)


# speedup vs baseline: 10.8340x; 1.0362x over previous
import functools

import jax
import jax.numpy as jnp
import numpy as np
from jax import lax
from jax.experimental import pallas as pl
from jax.experimental.pallas import tpu as pltpu
from jax.experimental.pallas import tpu_sc as plsc

K = 64
B = 128
H = 32768
NC, NS, L = 2, 16, 16
NW = NC * NS
ROWS_PER_W = B // NW
NVREG = H // L
HEAD_VREGS = 16
CHUNK = 127
NCHUNK = (NVREG - HEAD_VREGS) // CHUNK
REBUILD_AT = 560
CAP = 2624

INT_MIN = np.int32(-2147483648)
MANT = np.int32(0x7FFFFFFF)

parallel_loop = plsc.parallel_loop


def _iota():
    return lax.iota(jnp.int32, L)


def _pcount(mask):
    return plsc.all_reduce_population_count(mask)[0]


def _keys_of(v):
    u = plsc.bitcast(v, jnp.int32)
    return u ^ ((u >> 31) & MANT)


def _axidx(name):
    return lax.axis_index(name)


def _body(x_hbm, o_hbm, buf0, buf1, zbuf, winbuf, cand_i, cand_k,
          sem_in, sem_out):
    wid = _axidx("c") * NS + _axidx("s")
    zeros = jnp.zeros((L,), jnp.float32)
    bufs = [buf0, buf1]

    @parallel_loop(0, H, L, unroll=8, carry=jnp.int32(0))
    def _z(off, c):
        zbuf[pl.ds(off, L)] = zeros
        return c

    def rebuild(buf, cnt):
        mv = (cnt + L - 1) // L

        @parallel_loop(0, mv * L, L, unroll=2, carry=jnp.int32(0))
        def _keys(off, c):
            idx = cand_i[pl.ds(off, L)]
            valid = (_iota() + off) < cnt
            idxs = jnp.where(valid, idx, 0)
            v = plsc.load_gather(buf, [idxs])
            k = jnp.where(valid, _keys_of(v), INT_MIN)
            cand_k[pl.ds(off, L)] = k
            return c

        def search_it(_, lohi):
            lo, hi = lohi
            mid = (lo >> 1) + (hi >> 1) + ((lo & 1) | (hi & 1))

            @parallel_loop(0, mv * L, L, unroll=4,
                           carry=jnp.zeros((L,), jnp.int32))
            def acc(off, a):
                k = cand_k[pl.ds(off, L)]
                return a + (k >= mid).astype(jnp.int32)
            pred = jnp.sum(acc) >= K
            lo = jnp.where(pred, mid, lo)
            hi = jnp.where(pred, hi, mid - jnp.int32(1))
            return lo, hi
        v_key, _ = lax.fori_loop(
            0, 32, search_it, (INT_MIN, jnp.int32(2147483647)))

        @parallel_loop(0, mv * L, L, unroll=4,
                       carry=jnp.zeros((L,), jnp.int32))
        def gacc(off, a):
            k = cand_k[pl.ds(off, L)]
            return a + (k > v_key).astype(jnp.int32)
        cnt_gt = jnp.sum(gacc)
        allow = jnp.int32(K) - cnt_gt

        def comp_j(j, carry):
            oc, tr = carry
            k = cand_k[pl.ds(j * L, L)]
            idx = cand_i[pl.ds(j * L, L)]
            gt = k > v_key
            tie = k == v_key
            tp = plsc.cumsum(tie.astype(jnp.int32)) + tr
            keep = gt | (tie & (tp <= allow))
            plsc.store_compressed(cand_i.at[pl.ds(oc, L)], idx, mask=keep)
            return oc + _pcount(keep), tr + _pcount(tie)
        lax.fori_loop(0, mv, comp_j, (jnp.int32(0), jnp.int32(0)))

        thr_bits = jnp.full((L,), v_key ^ ((v_key >> 31) & MANT), jnp.int32)
        return plsc.bitcast(thr_bits, jnp.float32)

    def select_row(buf):
        @parallel_loop(0, HEAD_VREGS * L, L, unroll=4, carry=jnp.int32(0))
        def _seed(off, c):
            cand_i[pl.ds(off, L)] = _iota() + off
            return c
        thr = rebuild(buf, jnp.int32(HEAD_VREGS * L))
        cnt = jnp.int32(K)

        def chunk_c(c, carry):
            cnt, thr = carry
            base = HEAD_VREGS + c * CHUNK

            @parallel_loop(base * L, (base + CHUNK) * L, L,
                           unroll=4, carry=cnt)
            def cnt(off, cnt):
                v = buf[pl.ds(off, L)]
                m = v > thr
                plsc.store_compressed(
                    cand_i.at[pl.ds(cnt, L)], _iota() + off, mask=m)
                return cnt + _pcount(m)

            def do_rb(cnt):
                return jnp.int32(K), rebuild(buf, cnt)

            cnt, thr = lax.cond(
                cnt > REBUILD_AT, do_rb, lambda c: (c, thr), cnt)
            return cnt, thr
        cnt, _ = lax.fori_loop(0, NCHUNK, chunk_c, (cnt, thr))

        rebuild(buf, cnt)

    base_row = wid * ROWS_PER_W
    pltpu.async_copy(x_hbm.at[base_row], bufs[0], sem_in)
    for t in range(ROWS_PER_W):
        buf = bufs[t % 2]
        row = base_row + t
        pltpu.make_async_copy(x_hbm.at[row], buf, sem_in).wait()
        if t + 1 < ROWS_PER_W:
            pltpu.async_copy(x_hbm.at[row + 1], bufs[(t + 1) % 2], sem_in)

        select_row(buf)

        if t >= 1:
            pltpu.make_async_copy(zbuf, o_hbm.at[row - 1], sem_out).wait()
            for j in range(K // L):
                idx = winbuf[pl.ds(j * L, L)]
                plsc.store_scatter(zbuf, [idx], zeros)
        for j in range(K // L):
            idx = cand_i[pl.ds(j * L, L)]
            vals = plsc.load_gather(buf, [idx])
            plsc.store_scatter(zbuf, [idx], vals)
            winbuf[pl.ds(j * L, L)] = idx
        pltpu.async_copy(zbuf, o_hbm.at[row], sem_out)
    pltpu.make_async_copy(
        zbuf, o_hbm.at[base_row + ROWS_PER_W - 1], sem_out).wait()


@jax.jit
def kernel(hidden_preactivation_BH):
    mesh = plsc.VectorSubcoreMesh(
        core_axis_name="c", subcore_axis_name="s",
        num_cores=NC, num_subcores=NS)
    return pl.kernel(
        _body,
        out_type=jax.ShapeDtypeStruct((B, H), jnp.float32),
        mesh=mesh,
        scratch_types=[
            pltpu.VMEM((H,), jnp.float32),
            pltpu.VMEM((H,), jnp.float32),
            pltpu.VMEM((H,), jnp.float32),
            pltpu.VMEM((K,), jnp.int32),
            pltpu.VMEM((CAP,), jnp.int32),
            pltpu.VMEM((CAP,), jnp.int32),
            pltpu.SemaphoreType.DMA,
            pltpu.SemaphoreType.DMA,
        ],
        compiler_params=pltpu.CompilerParams(needs_layout_passes=False),
    )(hidden_preactivation_BH)
